# baseline (speedup 1.0000x reference)
import functools
import jax
import jax.numpy as jnp
from jax import lax
from jax.experimental import pallas as pl
from jax.experimental.pallas import tpu as pltpu
from jax.experimental.pallas import tpu_sc as plsc

B, N = 256, 64
EMB = 64
D_INV = 256

INV_BLK = 32
A_VOCAB = 128
C_VOCAB = 8

NPAIR = N * N
NW = 32
PAIRS_PER_W = NPAIR // NW
TILE_F32 = EMB * B
L = 16


def _onehot_t(idx_row, vocab):
    io = lax.broadcasted_iota(jnp.int32, (vocab, idx_row.shape[1]), 0)
    return (io == idx_row).astype(jnp.float32)


_TDOT = (((0,), (0,)), ((), ()))


def _invs_body(at_ref, ac_ref, ta_ref, tc_ref, w1a_ref, w1c_ref, b1_ref,
               w2_ref, b2_ref, out_ref):
    emb_a = []
    emb_c = []
    for i in range(INV_BLK):
        oh_a = _onehot_t(at_ref[i:i + 1, :], A_VOCAB)
        emb_a.append(lax.dot_general(oh_a, ta_ref[...], _TDOT,
                                     preferred_element_type=jnp.float32))
        oh_c = _onehot_t(ac_ref[i:i + 1, :], C_VOCAB)
        emb_c.append(lax.dot_general(oh_c, tc_ref[...], _TDOT,
                                     preferred_element_type=jnp.float32))
    ea = jnp.concatenate(emb_a, axis=0)
    ec = jnp.concatenate(emb_c, axis=0)
    h = (jnp.dot(ea, w1a_ref[...], preferred_element_type=jnp.float32)
         + jnp.dot(ec, w1c_ref[...], preferred_element_type=jnp.float32)
         + b1_ref[...])
    h = h * jax.nn.sigmoid(h)
    out = (jnp.dot(h, w2_ref[...], preferred_element_type=jnp.float32)
           + b2_ref[...])
    out_ref[...] = out.reshape(INV_BLK, 64, D_INV)


def _edges_sc(bt_hbm, tbl_hbm, out_hbm, bt_v, tbl_v, buf0, buf1, sem_in,
              sem0, sem1):
    wid = lax.axis_index("s") * 2 + lax.axis_index("c")
    pair0 = wid * PAIRS_PER_W

    pltpu.async_copy(tbl_hbm, tbl_v, sem_in).wait()
    pltpu.async_copy(
        bt_hbm.at[pl.ds(pair0 * B, PAIRS_PER_W * B)], bt_v, sem_in).wait()

    def compute_tile(p, buf):
        base = [bt_v[pl.ds(p * B + c * L, L)] * EMB for c in range(B // L)]

        def dbody(d, carry):
            dv = jnp.full((L,), d, jnp.int32)
            for c in range(B // L):
                row = plsc.load_gather(tbl_v, [base[c] + dv])
                buf[pl.ds(d * B + c * L, L)] = row
            return carry

        lax.fori_loop(0, EMB, dbody, 0, unroll=2)

    def out_slice(p):
        return out_hbm.at[pl.ds((pair0 + p) * TILE_F32, TILE_F32)]

    def pair_body(k, carry):
        p0 = 2 * k
        p1 = 2 * k + 1

        @pl.when(k > 0)
        def _():
            pltpu.make_async_copy(buf0, out_slice(p0 - 2), sem0).wait()

        compute_tile(p0, buf0)
        pltpu.make_async_copy(buf0, out_slice(p0), sem0).start()

        @pl.when(k > 0)
        def _():
            pltpu.make_async_copy(buf1, out_slice(p1 - 2), sem1).wait()

        compute_tile(p1, buf1)
        pltpu.make_async_copy(buf1, out_slice(p1), sem1).start()
        return carry

    lax.fori_loop(0, PAIRS_PER_W // 2, pair_body, 0)
    pltpu.make_async_copy(buf0, out_slice(PAIRS_PER_W - 2), sem0).wait()
    pltpu.make_async_copy(buf1, out_slice(PAIRS_PER_W - 1), sem1).wait()


def kernel(atom_types, bond_types, atom_mask, atom_charges, atom_type_table,
           charge_table, bond_table, W1, b1, W2, b2):
    del atom_mask

    ta = jnp.pad(atom_type_table, ((0, A_VOCAB - atom_type_table.shape[0]), (0, 0)))
    tc = jnp.pad(charge_table, ((0, C_VOCAB - charge_table.shape[0]), (0, 0)))
    w1a, w1c = W1[:EMB], W1[EMB:]
    b1r = b1.reshape(1, D_INV)
    b2r = b2.reshape(1, D_INV)

    full = lambda shape: pl.BlockSpec(shape, lambda i: (0,) * len(shape))
    invs = pl.pallas_call(
        _invs_body,
        grid=(B // INV_BLK,),
        in_specs=[
            pl.BlockSpec((INV_BLK, 64), lambda i: (i, 0)),
            pl.BlockSpec((INV_BLK, 64), lambda i: (i, 0)),
            full((A_VOCAB, EMB)),
            full((C_VOCAB, EMB)),
            full((EMB, D_INV)),
            full((EMB, D_INV)),
            full((1, D_INV)),
            full((D_INV, D_INV)),
            full((1, D_INV)),
        ],
        out_specs=pl.BlockSpec((INV_BLK, 64, D_INV), lambda i: (i, 0, 0)),
        out_shape=jax.ShapeDtypeStruct((B, N, D_INV), jnp.float32),
    )(atom_types, atom_charges, ta, tc, w1a, w1c, b1r, W2, b2r)

    btt = jnp.transpose(bond_types, (1, 2, 0)).reshape(NPAIR * B)
    tbl_flat = bond_table.reshape(8 * EMB)
    mesh = plsc.VectorSubcoreMesh(core_axis_name="c", subcore_axis_name="s")
    sc_call = functools.partial(
        pl.kernel,
        mesh=mesh,
        compiler_params=pltpu.CompilerParams(needs_layout_passes=False),
        out_type=jax.ShapeDtypeStruct((NPAIR * TILE_F32,), jnp.float32),
        scratch_types=[
            pltpu.VMEM((PAIRS_PER_W * B,), jnp.int32),
            pltpu.VMEM((8 * EMB,), jnp.float32),
            pltpu.VMEM((TILE_F32,), jnp.float32),
            pltpu.VMEM((TILE_F32,), jnp.float32),
            pltpu.SemaphoreType.DMA,
            pltpu.SemaphoreType.DMA,
            pltpu.SemaphoreType.DMA,
        ],
    )(_edges_sc)
    edges_t = sc_call(btt, tbl_flat).reshape(N, N, EMB, B)
    edges = jnp.transpose(edges_t, (3, 0, 1, 2))
    return invs, edges
